# baseline (speedup 1.0000x reference)
import functools

import jax
import jax.numpy as jnp
from jax import lax
from jax.experimental import pallas as pl
from jax.experimental.pallas import tpu as pltpu
from jax.experimental.pallas import tpu_sc as plsc

N_DATA = 8388608
N_QUERY = 4194304
NC = 2
NS = 16
NW = NC * NS
L = 16

N_BINS = 32768
BIN_SHIFT = 17
FRAC_MASK = (1 << BIN_SHIFT) - 1
HCHUNK = 16384
QCHUNK = 8192

_INT_MIN = -2147483648


def _to_ukey(x):
    b = lax.bitcast_convert_type(x, jnp.int32)
    m = (b >> 31) | jnp.full(b.shape, _INT_MIN, jnp.int32)
    return b ^ m


def _bin_of(ukey):
    return lax.shift_right_logical(ukey, BIN_SHIFT)


def _worker_id():
    return lax.axis_index("s") * NC + lax.axis_index("c")


def _hist_kernel(x_hbm, hist_hbm, chunk0, chunk1, hist_vmem, sem0, sem1):
    wid = _worker_id()
    n_per = N_DATA // NW
    n_chunks = n_per // HCHUNK
    groups = HCHUNK // L
    sems = [sem0, sem1]
    chunks = [chunk0, chunk1]

    ZUNROLL = 8
    zeros = jnp.zeros((L,), jnp.int32)

    def zero_body(i, _):
        for u in range(ZUNROLL):
            hist_vmem[pl.ds((i * ZUNROLL + u) * L, L)] = zeros
        return 0

    lax.fori_loop(0, N_BINS // (L * ZUNROLL), zero_body, 0)

    base = wid * n_per

    UNROLL = 16
    ones = jnp.ones((L,), jnp.int32)

    def src(k):
        return x_hbm.at[pl.ds(base + k * HCHUNK, HCHUNK)]

    def process(buf):
        def group_body(g, _):
            all_bins = []
            for u in range(UNROLL):
                x = chunks[buf][pl.ds((g * UNROLL + u) * L, L)]
                all_bins.append(_bin_of(_to_ukey(x)))
            for bins in all_bins:
                plsc.addupdate_scatter(hist_vmem, [bins], ones)
            return 0

        lax.fori_loop(0, groups // UNROLL, group_body, 0)

    pltpu.async_copy(src(0), chunk0, sem0)
    pltpu.async_copy(src(1), chunk1, sem1)

    def pair_body(p, _):
        for b in range(2):
            k = p * 2 + b
            pltpu.make_async_copy(src(k), chunks[b], sems[b]).wait()
            process(b)
            pltpu.async_copy(src(k + 2), chunks[b], sems[b])
        return 0

    lax.fori_loop(0, n_chunks // 2 - 1, pair_body, 0)
    for b in range(2):
        k = n_chunks - 2 + b
        pltpu.make_async_copy(src(k), chunks[b], sems[b]).wait()
        process(b)

    pltpu.sync_copy(hist_vmem, hist_hbm.at[wid])


def _cdf_kernel(hist_ref, out_ref):
    h = hist_ref[...].astype(jnp.float32)
    s = jnp.sum(h, axis=0)

    ii = lax.broadcasted_iota(jnp.int32, (128, 128), 0)
    jj = lax.broadcasted_iota(jnp.int32, (128, 128), 1)
    strict_upper = (ii < jj).astype(jnp.float32)
    row_excl = jnp.dot(s, strict_upper, precision=lax.Precision.HIGHEST)

    ri = lax.broadcasted_iota(jnp.int32, (256, 256), 0)
    rj = lax.broadcasted_iota(jnp.int32, (256, 256), 1)
    strict_lower = (rj < ri).astype(jnp.float32)
    row_tot = jnp.sum(s, axis=1, keepdims=True)
    row_off = jnp.dot(strict_lower, row_tot, precision=lax.Precision.HIGHEST)

    excl = row_excl + row_off
    incl = excl + s
    inv_n = jnp.float32(1.0 / N_DATA)
    out_ref[...] = jnp.concatenate([excl, incl], axis=0) * inv_n


def _lookup_kernel(time_hbm, table_hbm, out_hbm, te_vmem, ti_vmem, in0, in1,
                   out0, out1, sem_in0, sem_in1, sem_out0, sem_out1):
    wid = _worker_id()
    q_per = N_QUERY // NW
    n_chunks = q_per // QCHUNK
    groups = QCHUNK // L
    inv_bin = jnp.float32(1.0 / (1 << BIN_SHIFT))
    sems_in = [sem_in0, sem_in1]
    sems_out = [sem_out0, sem_out1]
    ins = [in0, in1]
    outs = [out0, out1]

    pltpu.sync_copy(table_hbm.at[pl.ds(0, N_BINS)], te_vmem)
    pltpu.sync_copy(table_hbm.at[pl.ds(N_BINS, N_BINS)], ti_vmem)
    base = wid * q_per

    UNROLL = 16

    def src(k):
        return time_hbm.at[pl.ds(base + k * QCHUNK, QCHUNK)]

    def dst(k):
        return out_hbm.at[pl.ds(base + k * QCHUNK, QCHUNK)]

    def batch_vals(buf, g):
        vals = []
        for u in range(UNROLL):
            off = (g * UNROLL + u) * L
            ukey = _to_ukey(ins[buf][pl.ds(off, L)])
            bins = _bin_of(ukey)
            frac = (ukey & FRAC_MASK).astype(jnp.float32) * inv_bin
            lo = plsc.load_gather(te_vmem, [bins])
            hi = plsc.load_gather(ti_vmem, [bins])
            vals.append(lo + (hi - lo) * frac)
        return vals

    def store_vals(buf, g, vals):
        for u, v in enumerate(vals):
            outs[buf][pl.ds((g * UNROLL + u) * L, L)] = v

    def process(buf):
        def group_body(g, carry):
            vals = batch_vals(buf, g + 1)
            store_vals(buf, g, carry)
            return vals

        first = batch_vals(buf, 0)
        last = lax.fori_loop(0, groups // UNROLL - 1, group_body, first)
        store_vals(buf, groups // UNROLL - 1, last)

    pltpu.async_copy(src(0), in0, sem_in0)
    pltpu.async_copy(src(1), in1, sem_in1)

    def pair_body(p, _):
        for b in range(2):
            k = p * 2 + b
            pltpu.make_async_copy(src(k), ins[b], sems_in[b]).wait()

            @pl.when(p > 0)
            def _():
                pltpu.make_async_copy(outs[b], dst(k), sems_out[b]).wait()

            process(b)
            pltpu.async_copy(src(k + 2), ins[b], sems_in[b])
            pltpu.async_copy(outs[b], dst(k), sems_out[b])
        return 0

    lax.fori_loop(0, n_chunks // 2 - 1, pair_body, 0)
    for b in range(2):
        k = n_chunks - 2 + b
        pltpu.make_async_copy(src(k), ins[b], sems_in[b]).wait()
        pltpu.make_async_copy(outs[b], dst(k), sems_out[b]).wait()
        process(b)
        pltpu.async_copy(outs[b], dst(k), sems_out[b])
    for b in range(2):
        k = n_chunks - 2 + b
        pltpu.make_async_copy(outs[b], dst(k), sems_out[b]).wait()


_SC_MESH = plsc.VectorSubcoreMesh(core_axis_name="c", subcore_axis_name="s")

_hist_call = functools.partial(
    pl.kernel,
    out_type=jax.ShapeDtypeStruct((NW, N_BINS), jnp.int32),
    mesh=_SC_MESH,
    scratch_types=[
        pltpu.VMEM((HCHUNK,), jnp.float32),
        pltpu.VMEM((HCHUNK,), jnp.float32),
        pltpu.VMEM((N_BINS,), jnp.int32),
        pltpu.SemaphoreType.DMA,
        pltpu.SemaphoreType.DMA,
    ],
    compiler_params=pltpu.CompilerParams(needs_layout_passes=False),
)(_hist_kernel)

_lookup_call = functools.partial(
    pl.kernel,
    out_type=jax.ShapeDtypeStruct((N_QUERY,), jnp.float32),
    mesh=_SC_MESH,
    scratch_types=[
        pltpu.VMEM((N_BINS,), jnp.float32),
        pltpu.VMEM((N_BINS,), jnp.float32),
        pltpu.VMEM((QCHUNK,), jnp.float32),
        pltpu.VMEM((QCHUNK,), jnp.float32),
        pltpu.VMEM((QCHUNK,), jnp.float32),
        pltpu.VMEM((QCHUNK,), jnp.float32),
        pltpu.SemaphoreType.DMA,
        pltpu.SemaphoreType.DMA,
        pltpu.SemaphoreType.DMA,
        pltpu.SemaphoreType.DMA,
    ],
    compiler_params=pltpu.CompilerParams(needs_layout_passes=False),
)(_lookup_kernel)

_cdf_call = pl.pallas_call(
    _cdf_kernel,
    out_shape=jax.ShapeDtypeStruct((512, 128), jnp.float32),
)


def kernel(time, x_data):
    hist = _hist_call(x_data)
    table = _cdf_call(hist.reshape(NW, 256, 128))
    return _lookup_call(time, table.reshape(2 * N_BINS))


# speedup vs baseline: 1.0192x; 1.0192x over previous
import functools

import jax
import jax.numpy as jnp
from jax import lax
from jax.experimental import pallas as pl
from jax.experimental.pallas import tpu as pltpu
from jax.experimental.pallas import tpu_sc as plsc

N_DATA = 8388608
N_QUERY = 4194304
NC = 2
NS = 16
NW = NC * NS
L = 16

N_BINS = 32768
BIN_SHIFT = 17
FRAC_MASK = (1 << BIN_SHIFT) - 1
HCHUNK = 16384
QCHUNK = 8192

_INT_MIN = -2147483648


def _to_ukey(x):
    b = lax.bitcast_convert_type(x, jnp.int32)
    m = (b >> 31) | jnp.full(b.shape, _INT_MIN, jnp.int32)
    return b ^ m


def _bin_of(ukey):
    return lax.shift_right_logical(ukey, BIN_SHIFT)


def _worker_id():
    return lax.axis_index("s") * NC + lax.axis_index("c")


def _hist_kernel(x_hbm, hist_hbm, chunk0, chunk1, hist_vmem, sem0, sem1):
    wid = _worker_id()
    n_per = N_DATA // NW
    n_chunks = n_per // HCHUNK
    groups = HCHUNK // L
    sems = [sem0, sem1]
    chunks = [chunk0, chunk1]

    ZUNROLL = 8
    zeros = jnp.zeros((L,), jnp.int32)

    def zero_body(i, _):
        for u in range(ZUNROLL):
            hist_vmem[pl.ds((i * ZUNROLL + u) * L, L)] = zeros
        return 0

    lax.fori_loop(0, N_BINS // (L * ZUNROLL), zero_body, 0)

    base = wid * n_per

    UNROLL = 16
    ones = jnp.ones((L,), jnp.int32)

    def src(k):
        return x_hbm.at[pl.ds(base + k * HCHUNK, HCHUNK)]

    def process(buf):
        def group_body(g, _):
            all_bins = []
            for u in range(UNROLL):
                x = chunks[buf][pl.ds((g * UNROLL + u) * L, L)]
                all_bins.append(_bin_of(_to_ukey(x)))
            for bins in all_bins:
                plsc.addupdate_scatter(hist_vmem, [bins], ones)
            return 0

        lax.fori_loop(0, groups // UNROLL, group_body, 0)

    pltpu.async_copy(src(0), chunk0, sem0)
    pltpu.async_copy(src(1), chunk1, sem1)

    def pair_body(p, _):
        for b in range(2):
            k = p * 2 + b
            pltpu.make_async_copy(src(k), chunks[b], sems[b]).wait()
            process(b)
            pltpu.async_copy(src(k + 2), chunks[b], sems[b])
        return 0

    lax.fori_loop(0, n_chunks // 2 - 1, pair_body, 0)
    for b in range(2):
        k = n_chunks - 2 + b
        pltpu.make_async_copy(src(k), chunks[b], sems[b]).wait()
        process(b)

    pltpu.sync_copy(hist_vmem, hist_hbm.at[wid])


def _cdf_kernel(hist_ref, out_ref):
    h = hist_ref[...].astype(jnp.float32)
    s = jnp.sum(h, axis=0)

    ii = lax.broadcasted_iota(jnp.int32, (128, 128), 0)
    jj = lax.broadcasted_iota(jnp.int32, (128, 128), 1)
    strict_upper = (ii < jj).astype(jnp.float32)
    row_excl = jnp.dot(s, strict_upper, precision=lax.Precision.HIGHEST)

    ri = lax.broadcasted_iota(jnp.int32, (256, 256), 0)
    rj = lax.broadcasted_iota(jnp.int32, (256, 256), 1)
    strict_lower = (rj < ri).astype(jnp.float32)
    row_tot = jnp.sum(s, axis=1, keepdims=True)
    row_off = jnp.dot(strict_lower, row_tot, precision=lax.Precision.HIGHEST)

    excl = row_excl + row_off
    incl = excl + s
    inv_n = jnp.float32(1.0 / N_DATA)
    out_ref[...] = jnp.concatenate([excl, incl], axis=0) * inv_n


def _lookup_kernel(time_hbm, table_hbm, out_hbm, te_vmem, ti_vmem, in0, in1,
                   out0, out1, sem_in0, sem_in1, sem_out0, sem_out1):
    wid = _worker_id()
    q_per = N_QUERY // NW
    n_chunks = q_per // QCHUNK
    groups = QCHUNK // L
    inv_bin = jnp.float32(1.0 / (1 << BIN_SHIFT))
    sems_in = [sem_in0, sem_in1]
    sems_out = [sem_out0, sem_out1]
    ins = [in0, in1]
    outs = [out0, out1]

    pltpu.sync_copy(table_hbm.at[pl.ds(0, N_BINS)], te_vmem)
    pltpu.sync_copy(table_hbm.at[pl.ds(N_BINS, N_BINS)], ti_vmem)
    base = wid * q_per

    UNROLL = 16

    def src(k):
        return time_hbm.at[pl.ds(base + k * QCHUNK, QCHUNK)]

    def dst(k):
        return out_hbm.at[pl.ds(base + k * QCHUNK, QCHUNK)]

    def batch_vals(buf, g):
        vals = []
        for u in range(UNROLL):
            off = (g * UNROLL + u) * L
            ukey = _to_ukey(ins[buf][pl.ds(off, L)])
            bins = _bin_of(ukey)
            frac = (ukey & FRAC_MASK).astype(jnp.float32) * inv_bin
            lo = plsc.load_gather(te_vmem, [bins])
            hi = plsc.load_gather(ti_vmem, [bins])
            vals.append(lo + (hi - lo) * frac)
        return vals

    def store_vals(buf, g, vals):
        for u, v in enumerate(vals):
            outs[buf][pl.ds((g * UNROLL + u) * L, L)] = v

    def process(buf):
        def group_body(g, _):
            store_vals(buf, g, batch_vals(buf, g))
            return 0

        lax.fori_loop(0, groups // UNROLL, group_body, 0)

    pltpu.async_copy(src(0), in0, sem_in0)
    pltpu.async_copy(src(1), in1, sem_in1)

    def pair_body(p, _):
        for b in range(2):
            k = p * 2 + b
            pltpu.make_async_copy(src(k), ins[b], sems_in[b]).wait()

            @pl.when(p > 0)
            def _():
                pltpu.make_async_copy(outs[b], dst(k), sems_out[b]).wait()

            process(b)
            pltpu.async_copy(src(k + 2), ins[b], sems_in[b])
            pltpu.async_copy(outs[b], dst(k), sems_out[b])
        return 0

    lax.fori_loop(0, n_chunks // 2 - 1, pair_body, 0)
    for b in range(2):
        k = n_chunks - 2 + b
        pltpu.make_async_copy(src(k), ins[b], sems_in[b]).wait()
        pltpu.make_async_copy(outs[b], dst(k), sems_out[b]).wait()
        process(b)
        pltpu.async_copy(outs[b], dst(k), sems_out[b])
    for b in range(2):
        k = n_chunks - 2 + b
        pltpu.make_async_copy(outs[b], dst(k), sems_out[b]).wait()


_SC_MESH = plsc.VectorSubcoreMesh(core_axis_name="c", subcore_axis_name="s")

_hist_call = functools.partial(
    pl.kernel,
    out_type=jax.ShapeDtypeStruct((NW, N_BINS), jnp.int32),
    mesh=_SC_MESH,
    scratch_types=[
        pltpu.VMEM((HCHUNK,), jnp.float32),
        pltpu.VMEM((HCHUNK,), jnp.float32),
        pltpu.VMEM((N_BINS,), jnp.int32),
        pltpu.SemaphoreType.DMA,
        pltpu.SemaphoreType.DMA,
    ],
    compiler_params=pltpu.CompilerParams(needs_layout_passes=False),
)(_hist_kernel)

_lookup_call = functools.partial(
    pl.kernel,
    out_type=jax.ShapeDtypeStruct((N_QUERY,), jnp.float32),
    mesh=_SC_MESH,
    scratch_types=[
        pltpu.VMEM((N_BINS,), jnp.float32),
        pltpu.VMEM((N_BINS,), jnp.float32),
        pltpu.VMEM((QCHUNK,), jnp.float32),
        pltpu.VMEM((QCHUNK,), jnp.float32),
        pltpu.VMEM((QCHUNK,), jnp.float32),
        pltpu.VMEM((QCHUNK,), jnp.float32),
        pltpu.SemaphoreType.DMA,
        pltpu.SemaphoreType.DMA,
        pltpu.SemaphoreType.DMA,
        pltpu.SemaphoreType.DMA,
    ],
    compiler_params=pltpu.CompilerParams(needs_layout_passes=False),
)(_lookup_kernel)

_cdf_call = pl.pallas_call(
    _cdf_kernel,
    out_shape=jax.ShapeDtypeStruct((512, 128), jnp.float32),
)


def kernel(time, x_data):
    hist = _hist_call(x_data)
    table = _cdf_call(hist.reshape(NW, 256, 128))
    return _lookup_call(time, table.reshape(2 * N_BINS))


# speedup vs baseline: 1.0782x; 1.0579x over previous
import functools

import jax
import jax.numpy as jnp
from jax import lax
from jax.experimental import pallas as pl
from jax.experimental.pallas import tpu as pltpu
from jax.experimental.pallas import tpu_sc as plsc

N_DATA = 8388608
N_QUERY = 4194304
NC = 2
NS = 16
NW = NC * NS
L = 16

N_BINS = 32768
BIN_SHIFT = 17
FRAC_MASK = (1 << BIN_SHIFT) - 1
HCHUNK = 16384
QCHUNK = 8192

_INT_MIN = -2147483648


def _to_ukey(x):
    b = lax.bitcast_convert_type(x, jnp.int32)
    m = (b >> 31) | jnp.full(b.shape, _INT_MIN, jnp.int32)
    return b ^ m


def _bin_of(ukey):
    return lax.shift_right_logical(ukey, BIN_SHIFT)


def _worker_id():
    return lax.axis_index("s") * NC + lax.axis_index("c")


def _hist_kernel(x_hbm, hist_hbm, chunk0, chunk1, hist_vmem, sem0, sem1):
    wid = _worker_id()
    n_per = N_DATA // NW
    n_chunks = n_per // HCHUNK
    groups = HCHUNK // L
    sems = [sem0, sem1]
    chunks = [chunk0, chunk1]

    ZUNROLL = 8
    zeros = jnp.zeros((L,), jnp.int32)

    def zero_body(i, _):
        for u in range(ZUNROLL):
            hist_vmem[pl.ds((i * ZUNROLL + u) * L, L)] = zeros
        return 0

    lax.fori_loop(0, N_BINS // (L * ZUNROLL), zero_body, 0)

    base = wid * n_per

    UNROLL = 16
    ones = jnp.ones((L,), jnp.int32)

    def src(k):
        return x_hbm.at[pl.ds(base + k * HCHUNK, HCHUNK)]

    def process(buf):
        def group_body(g, _):
            all_bins = []
            for u in range(UNROLL):
                x = chunks[buf][pl.ds((g * UNROLL + u) * L, L)]
                all_bins.append(_bin_of(_to_ukey(x)))
            for bins in all_bins:
                plsc.addupdate_scatter(hist_vmem, [bins], ones)
            return 0

        lax.fori_loop(0, groups // UNROLL, group_body, 0)

    pltpu.async_copy(src(0), chunk0, sem0)
    pltpu.async_copy(src(1), chunk1, sem1)

    def pair_body(p, _):
        for b in range(2):
            k = p * 2 + b
            pltpu.make_async_copy(src(k), chunks[b], sems[b]).wait()
            process(b)
            pltpu.async_copy(src(k + 2), chunks[b], sems[b])
        return 0

    lax.fori_loop(0, n_chunks // 2 - 1, pair_body, 0)
    for b in range(2):
        k = n_chunks - 2 + b
        pltpu.make_async_copy(src(k), chunks[b], sems[b]).wait()
        process(b)

    pltpu.sync_copy(hist_vmem, hist_hbm.at[wid])


def _cdf_kernel(hist_ref, out_ref):
    h = hist_ref[...].astype(jnp.float32)
    s = jnp.sum(h, axis=0).reshape(256, 128)

    ii = lax.broadcasted_iota(jnp.int32, (128, 128), 0)
    jj = lax.broadcasted_iota(jnp.int32, (128, 128), 1)
    strict_upper = (ii < jj).astype(jnp.float32)
    row_excl = jnp.dot(s, strict_upper, precision=lax.Precision.HIGHEST)

    ri = lax.broadcasted_iota(jnp.int32, (256, 256), 0)
    rj = lax.broadcasted_iota(jnp.int32, (256, 256), 1)
    strict_lower = (rj < ri).astype(jnp.float32)
    row_tot = jnp.sum(s, axis=1, keepdims=True)
    row_off = jnp.dot(strict_lower, row_tot, precision=lax.Precision.HIGHEST)

    excl = row_excl + row_off
    incl = excl + s
    inv_n = jnp.float32(1.0 / N_DATA)
    out_ref[...] = jnp.concatenate([excl, incl], axis=0) * inv_n


def _lookup_kernel(time_hbm, table_hbm, out_hbm, te_vmem, ti_vmem, in0, in1,
                   out0, out1, sem_in0, sem_in1, sem_out0, sem_out1):
    wid = _worker_id()
    q_per = N_QUERY // NW
    n_chunks = q_per // QCHUNK
    groups = QCHUNK // L
    inv_bin = jnp.float32(1.0 / (1 << BIN_SHIFT))
    sems_in = [sem_in0, sem_in1]
    sems_out = [sem_out0, sem_out1]
    ins = [in0, in1]
    outs = [out0, out1]

    pltpu.sync_copy(table_hbm.at[pl.ds(0, N_BINS)], te_vmem)
    pltpu.sync_copy(table_hbm.at[pl.ds(N_BINS, N_BINS)], ti_vmem)
    base = wid * q_per

    UNROLL = 16

    def src(k):
        return time_hbm.at[pl.ds(base + k * QCHUNK, QCHUNK)]

    def dst(k):
        return out_hbm.at[pl.ds(base + k * QCHUNK, QCHUNK)]

    def batch_vals(buf, g):
        vals = []
        for u in range(UNROLL):
            off = (g * UNROLL + u) * L
            ukey = _to_ukey(ins[buf][pl.ds(off, L)])
            bins = _bin_of(ukey)
            frac = (ukey & FRAC_MASK).astype(jnp.float32) * inv_bin
            lo = plsc.load_gather(te_vmem, [bins])
            hi = plsc.load_gather(ti_vmem, [bins])
            vals.append(lo + (hi - lo) * frac)
        return vals

    def store_vals(buf, g, vals):
        for u, v in enumerate(vals):
            outs[buf][pl.ds((g * UNROLL + u) * L, L)] = v

    def process(buf):
        def group_body(g, _):
            store_vals(buf, g, batch_vals(buf, g))
            return 0

        lax.fori_loop(0, groups // UNROLL, group_body, 0)

    pltpu.async_copy(src(0), in0, sem_in0)
    pltpu.async_copy(src(1), in1, sem_in1)

    def pair_body(p, _):
        for b in range(2):
            k = p * 2 + b
            pltpu.make_async_copy(src(k), ins[b], sems_in[b]).wait()

            @pl.when(p > 0)
            def _():
                pltpu.make_async_copy(outs[b], dst(k), sems_out[b]).wait()

            process(b)
            pltpu.async_copy(src(k + 2), ins[b], sems_in[b])
            pltpu.async_copy(outs[b], dst(k), sems_out[b])
        return 0

    lax.fori_loop(0, n_chunks // 2 - 1, pair_body, 0)
    for b in range(2):
        k = n_chunks - 2 + b
        pltpu.make_async_copy(src(k), ins[b], sems_in[b]).wait()
        pltpu.make_async_copy(outs[b], dst(k), sems_out[b]).wait()
        process(b)
        pltpu.async_copy(outs[b], dst(k), sems_out[b])
    for b in range(2):
        k = n_chunks - 2 + b
        pltpu.make_async_copy(outs[b], dst(k), sems_out[b]).wait()


_SC_MESH = plsc.VectorSubcoreMesh(core_axis_name="c", subcore_axis_name="s")

_hist_call = functools.partial(
    pl.kernel,
    out_type=jax.ShapeDtypeStruct((NW, N_BINS), jnp.int32),
    mesh=_SC_MESH,
    scratch_types=[
        pltpu.VMEM((HCHUNK,), jnp.float32),
        pltpu.VMEM((HCHUNK,), jnp.float32),
        pltpu.VMEM((N_BINS,), jnp.int32),
        pltpu.SemaphoreType.DMA,
        pltpu.SemaphoreType.DMA,
    ],
    compiler_params=pltpu.CompilerParams(needs_layout_passes=False),
)(_hist_kernel)

_lookup_call = functools.partial(
    pl.kernel,
    out_type=jax.ShapeDtypeStruct((N_QUERY,), jnp.float32),
    mesh=_SC_MESH,
    scratch_types=[
        pltpu.VMEM((N_BINS,), jnp.float32),
        pltpu.VMEM((N_BINS,), jnp.float32),
        pltpu.VMEM((QCHUNK,), jnp.float32),
        pltpu.VMEM((QCHUNK,), jnp.float32),
        pltpu.VMEM((QCHUNK,), jnp.float32),
        pltpu.VMEM((QCHUNK,), jnp.float32),
        pltpu.SemaphoreType.DMA,
        pltpu.SemaphoreType.DMA,
        pltpu.SemaphoreType.DMA,
        pltpu.SemaphoreType.DMA,
    ],
    compiler_params=pltpu.CompilerParams(needs_layout_passes=False),
)(_lookup_kernel)

_cdf_call = pl.pallas_call(
    _cdf_kernel,
    out_shape=jax.ShapeDtypeStruct((512, 128), jnp.float32),
)


def kernel(time, x_data):
    hist = _hist_call(x_data)
    table = _cdf_call(hist)
    return _lookup_call(time, table.reshape(2 * N_BINS))


# speedup vs baseline: 1.0797x; 1.0013x over previous
import functools

import jax
import jax.numpy as jnp
from jax import lax
from jax.experimental import pallas as pl
from jax.experimental.pallas import tpu as pltpu
from jax.experimental.pallas import tpu_sc as plsc

N_DATA = 8388608
N_QUERY = 4194304
NC = 2
NS = 16
NW = NC * NS
L = 16

N_BINS = 32768
BIN_SHIFT = 17
FRAC_MASK = (1 << BIN_SHIFT) - 1
HCHUNK = 16384
QCHUNK = 8192

_INT_MIN = -2147483648


def _to_ukey(x):
    b = lax.bitcast_convert_type(x, jnp.int32)
    m = (b >> 31) | jnp.full(b.shape, _INT_MIN, jnp.int32)
    return b ^ m


def _bin_of(ukey):
    return lax.shift_right_logical(ukey, BIN_SHIFT)


def _worker_id():
    return lax.axis_index("s") * NC + lax.axis_index("c")


def _hist_kernel(x_hbm, hist_hbm, chunk0, chunk1, hist_vmem, sem0, sem1):
    wid = _worker_id()
    n_per = N_DATA // NW
    n_chunks = n_per // HCHUNK
    groups = HCHUNK // L
    sems = [sem0, sem1]
    chunks = [chunk0, chunk1]

    ZUNROLL = 8
    zeros = jnp.zeros((L,), jnp.int32)

    def zero_body(i, _):
        for u in range(ZUNROLL):
            hist_vmem[pl.ds((i * ZUNROLL + u) * L, L)] = zeros
        return 0

    lax.fori_loop(0, N_BINS // (L * ZUNROLL), zero_body, 0)

    base = wid * n_per

    UNROLL = 16
    ones = jnp.ones((L,), jnp.int32)

    def src(k):
        return x_hbm.at[pl.ds(base + k * HCHUNK, HCHUNK)]

    def process(buf):
        def group_body(g, _):
            all_bins = []
            for u in range(UNROLL):
                x = chunks[buf][pl.ds((g * UNROLL + u) * L, L)]
                all_bins.append(_bin_of(_to_ukey(x)))
            for bins in all_bins:
                plsc.addupdate_scatter(hist_vmem, [bins], ones)
            return 0

        lax.fori_loop(0, groups // UNROLL, group_body, 0)

    pltpu.async_copy(src(0), chunk0, sem0)
    pltpu.async_copy(src(1), chunk1, sem1)

    def pair_body(p, _):
        for b in range(2):
            k = p * 2 + b
            pltpu.make_async_copy(src(k), chunks[b], sems[b]).wait()
            process(b)
            pltpu.async_copy(src(k + 2), chunks[b], sems[b])
        return 0

    lax.fori_loop(0, n_chunks // 2 - 1, pair_body, 0)
    for b in range(2):
        k = n_chunks - 2 + b
        pltpu.make_async_copy(src(k), chunks[b], sems[b]).wait()
        process(b)

    pltpu.sync_copy(hist_vmem, hist_hbm.at[wid])


def _cdf_kernel(hist_ref, out_ref):
    h = hist_ref[...].astype(jnp.float32)
    s = jnp.sum(h, axis=0).reshape(256, 128)

    ii = lax.broadcasted_iota(jnp.int32, (128, 128), 0)
    jj = lax.broadcasted_iota(jnp.int32, (128, 128), 1)
    strict_upper = (ii < jj).astype(jnp.float32)
    row_excl = jnp.dot(s, strict_upper, precision=lax.Precision.HIGHEST)

    ri = lax.broadcasted_iota(jnp.int32, (256, 256), 0)
    rj = lax.broadcasted_iota(jnp.int32, (256, 256), 1)
    strict_lower = (rj < ri).astype(jnp.float32)
    row_tot = jnp.sum(s, axis=1, keepdims=True)
    row_off = jnp.dot(strict_lower, row_tot, precision=lax.Precision.HIGHEST)

    excl = row_excl + row_off
    incl = excl + s
    inv_n = jnp.float32(1.0 / N_DATA)
    out_ref[...] = (jnp.concatenate([excl, incl], axis=0) * inv_n).reshape(2 * N_BINS)


def _lookup_kernel(time_hbm, table_hbm, out_hbm, te_vmem, ti_vmem, in0, in1,
                   out0, out1, sem_in0, sem_in1, sem_out0, sem_out1):
    wid = _worker_id()
    q_per = N_QUERY // NW
    n_chunks = q_per // QCHUNK
    groups = QCHUNK // L
    inv_bin = jnp.float32(1.0 / (1 << BIN_SHIFT))
    sems_in = [sem_in0, sem_in1]
    sems_out = [sem_out0, sem_out1]
    ins = [in0, in1]
    outs = [out0, out1]

    pltpu.sync_copy(table_hbm.at[pl.ds(0, N_BINS)], te_vmem)
    pltpu.sync_copy(table_hbm.at[pl.ds(N_BINS, N_BINS)], ti_vmem)
    base = wid * q_per

    UNROLL = 16

    def src(k):
        return time_hbm.at[pl.ds(base + k * QCHUNK, QCHUNK)]

    def dst(k):
        return out_hbm.at[pl.ds(base + k * QCHUNK, QCHUNK)]

    def batch_vals(buf, g):
        vals = []
        for u in range(UNROLL):
            off = (g * UNROLL + u) * L
            ukey = _to_ukey(ins[buf][pl.ds(off, L)])
            bins = _bin_of(ukey)
            frac = (ukey & FRAC_MASK).astype(jnp.float32) * inv_bin
            lo = plsc.load_gather(te_vmem, [bins])
            hi = plsc.load_gather(ti_vmem, [bins])
            vals.append(lo + (hi - lo) * frac)
        return vals

    def store_vals(buf, g, vals):
        for u, v in enumerate(vals):
            outs[buf][pl.ds((g * UNROLL + u) * L, L)] = v

    def process(buf):
        def group_body(g, _):
            store_vals(buf, g, batch_vals(buf, g))
            return 0

        lax.fori_loop(0, groups // UNROLL, group_body, 0)

    pltpu.async_copy(src(0), in0, sem_in0)
    pltpu.async_copy(src(1), in1, sem_in1)

    def pair_body(p, _):
        for b in range(2):
            k = p * 2 + b
            pltpu.make_async_copy(src(k), ins[b], sems_in[b]).wait()

            @pl.when(p > 0)
            def _():
                pltpu.make_async_copy(outs[b], dst(k), sems_out[b]).wait()

            process(b)
            pltpu.async_copy(src(k + 2), ins[b], sems_in[b])
            pltpu.async_copy(outs[b], dst(k), sems_out[b])
        return 0

    lax.fori_loop(0, n_chunks // 2 - 1, pair_body, 0)
    for b in range(2):
        k = n_chunks - 2 + b
        pltpu.make_async_copy(src(k), ins[b], sems_in[b]).wait()
        pltpu.make_async_copy(outs[b], dst(k), sems_out[b]).wait()
        process(b)
        pltpu.async_copy(outs[b], dst(k), sems_out[b])
    for b in range(2):
        k = n_chunks - 2 + b
        pltpu.make_async_copy(outs[b], dst(k), sems_out[b]).wait()


_SC_MESH = plsc.VectorSubcoreMesh(core_axis_name="c", subcore_axis_name="s")

_hist_call = functools.partial(
    pl.kernel,
    out_type=jax.ShapeDtypeStruct((NW, N_BINS), jnp.int32),
    mesh=_SC_MESH,
    scratch_types=[
        pltpu.VMEM((HCHUNK,), jnp.float32),
        pltpu.VMEM((HCHUNK,), jnp.float32),
        pltpu.VMEM((N_BINS,), jnp.int32),
        pltpu.SemaphoreType.DMA,
        pltpu.SemaphoreType.DMA,
    ],
    compiler_params=pltpu.CompilerParams(needs_layout_passes=False),
)(_hist_kernel)

_lookup_call = functools.partial(
    pl.kernel,
    out_type=jax.ShapeDtypeStruct((N_QUERY,), jnp.float32),
    mesh=_SC_MESH,
    scratch_types=[
        pltpu.VMEM((N_BINS,), jnp.float32),
        pltpu.VMEM((N_BINS,), jnp.float32),
        pltpu.VMEM((QCHUNK,), jnp.float32),
        pltpu.VMEM((QCHUNK,), jnp.float32),
        pltpu.VMEM((QCHUNK,), jnp.float32),
        pltpu.VMEM((QCHUNK,), jnp.float32),
        pltpu.SemaphoreType.DMA,
        pltpu.SemaphoreType.DMA,
        pltpu.SemaphoreType.DMA,
        pltpu.SemaphoreType.DMA,
    ],
    compiler_params=pltpu.CompilerParams(needs_layout_passes=False),
)(_lookup_kernel)

_cdf_call = pl.pallas_call(
    _cdf_kernel,
    out_shape=jax.ShapeDtypeStruct((2 * N_BINS,), jnp.float32),
)


def kernel(time, x_data):
    hist = _hist_call(x_data)
    table = _cdf_call(hist)
    return _lookup_call(time, table)
